# baseline (speedup 1.0000x reference)
import functools

import jax
import jax.numpy as jnp
from jax import lax
from jax.experimental import pallas as pl
from jax.experimental.pallas import tpu as pltpu
from jax.experimental.pallas import tpu_sc as plsc

N = 10000
NPAD = 10240
E = 320000
H = 128
HW = H // 2
NW = 32
EPW = E // NW
KC = 40
NCHUNK = EPW // KC
NBLK = 2048
EBLK = 2560

_MESH = dict(core_axis_name="c", subcore_axis_name="s")


def _gather_combine(table, idxa, idxb, wo, diff):
    ngrp = wo // 16
    assert ngrp == 8 + (1 if diff else 0)

    def body(table_ref, ia_ref, ib_ref, out_ref,
             ia_v, ib_v, a0, b0, o0, a1, b1, o1, sg0, sg1, ss0, ss1):
        cid = lax.axis_index("c")
        sid = lax.axis_index("s")
        wid = sid * 2 + cid
        pltpu.sync_copy(ia_ref.at[wid], ia_v)
        pltpu.sync_copy(ib_ref.at[wid], ib_v)
        bufs = ((a0, b0, o0, sg0, ss0), (a1, b1, o1, sg1, ss1))

        def issue(j, p):
            a, b, _, sg, _ = bufs[p]
            pltpu.async_copy(table_ref.at[ia_v.at[j]], a, sg)
            pltpu.async_copy(table_ref.at[ib_v.at[j]], b, sg)

        def wait_gather(p):
            a, b, sg = bufs[p][0], bufs[p][1], bufs[p][3]
            pltpu.make_async_copy(table_ref.at[ia_v.at[0]], a, sg).wait()
            pltpu.make_async_copy(table_ref.at[ib_v.at[0]], b, sg).wait()

        def wait_store(p):
            o, ss = bufs[p][2], bufs[p][4]
            pltpu.make_async_copy(o, out_ref.at[pl.ds(0, KC)], ss).wait()

        def combine_store(j, p):
            a, b, o, _, ss = bufs[p]

            def fb(w):
                return lax.bitcast_convert_type(w, jnp.float32)

            def ib(x):
                return lax.bitcast_convert_type(x, jnp.int32)

            def rnd(u):
                return u + 32767 + ((u >> 16) & 1)

            def comb(wa, wb, sign):
                ae = fb(wa << 16)
                ao = fb(wa & -65536)
                be_ = fb(wb << 16)
                bo = fb(wb & -65536)
                if sign < 0:
                    se, so = be_ - ae, bo - ao
                else:
                    se, so = ae + be_, ao + bo
                re = lax.shift_right_logical(rnd(ib(se)), 16)
                ro = rnd(ib(so)) & -65536
                return re | ro

            def row(r, carry):
                for gr in range(ngrp):
                    sl = pl.ds(gr * 16, 16)
                    if gr < 4:
                        v = comb(a[r, sl], b[r, pl.ds(gr * 16 + 64, 16)], 1)
                    elif gr < 8:
                        v = comb(a[r, sl], b[r, pl.ds(gr * 16 - 64, 16)], 1)
                    else:
                        v = comb(a[r, sl], b[r, sl], -1)
                    o[r, sl] = v
                return carry

            lax.fori_loop(0, KC, row, 0)
            pltpu.async_copy(o, out_ref.at[pl.ds(wid * EPW + j * KC, KC)], ss)

        issue(0, 0)

        def step(j2, carry):
            issue(2 * j2 + 1, 1)
            wait_gather(0)

            @pl.when(j2 > 0)
            def _():
                wait_store(0)

            combine_store(2 * j2, 0)

            @pl.when(j2 + 1 < NCHUNK // 2)
            def _():
                issue(2 * j2 + 2, 0)

            wait_gather(1)

            @pl.when(j2 > 0)
            def _():
                wait_store(1)

            combine_store(2 * j2 + 1, 1)
            return carry

        lax.fori_loop(0, NCHUNK // 2, step, 0)
        wait_store(0)
        wait_store(1)

    wt = table.shape[1]
    gbuf = pltpu.VMEM((KC, wt), jnp.int32)
    obuf = pltpu.VMEM((KC, wo), jnp.int32)
    return pl.kernel(
        body,
        out_type=jax.ShapeDtypeStruct((E, wo), jnp.int32),
        mesh=plsc.VectorSubcoreMesh(**_MESH),
        scratch_types=(
            pltpu.VMEM((NCHUNK, KC), jnp.int32),
            pltpu.VMEM((NCHUNK, KC), jnp.int32),
            gbuf, gbuf, obuf, gbuf, gbuf, obuf,
            pltpu.SemaphoreType.DMA,
            pltpu.SemaphoreType.DMA,
            pltpu.SemaphoreType.DMA,
            pltpu.SemaphoreType.DMA,
        ),
        name=f"sc_gather_combine_{wt}_{wo}",
    )(table, idxa, idxb)


def _scatter_add(msg, idxd, zeros):
    def body(msg_ref, idx_ref, zeros_ref, out_ref,
             idx_v, m0, m1, sl0, sl1, ss0, ss1, aggr_sh):
        cid = lax.axis_index("c")
        sid = lax.axis_index("s")
        wid = sid * 2 + cid

        @pl.when(sid == 0)
        def _():
            pltpu.sync_copy(zeros_ref, aggr_sh)

        plsc.subcore_barrier()
        pltpu.sync_copy(idx_ref.at[wid], idx_v)
        bufs = ((m0, sl0, ss0), (m1, sl1, ss1))

        def load(j, p):
            m, sl, _ = bufs[p]
            pltpu.async_copy(msg_ref.at[pl.ds(wid * EPW + j * KC, KC)], m, sl)

        def wait_load(p):
            m, sl, _ = bufs[p]
            pltpu.make_async_copy(msg_ref.at[pl.ds(0, KC)], m, sl).wait()

        def scat(j, p):
            m, _, ss = bufs[p]
            pltpu.async_copy(m, aggr_sh.at[idx_v.at[j]], ss, add=True)

        def wait_scat(p):
            m, _, ss = bufs[p]
            pltpu.make_async_copy(m, aggr_sh.at[idx_v.at[0]], ss).wait()

        load(0, 0)

        def step(j2, carry):
            load(2 * j2 + 1, 1)
            wait_load(0)
            scat(2 * j2, 0)
            wait_scat(0)

            @pl.when(j2 + 1 < NCHUNK // 2)
            def _():
                load(2 * j2 + 2, 0)

            wait_load(1)
            scat(2 * j2 + 1, 1)
            wait_scat(1)
            return carry

        lax.fori_loop(0, NCHUNK // 2, step, 0)
        plsc.subcore_barrier()
        rows = NPAD // 16
        pltpu.sync_copy(aggr_sh.at[pl.ds(sid * rows, rows)],
                        out_ref.at[pl.ds(cid * NPAD + sid * rows, rows)])

    mbuf = pltpu.VMEM((KC, H), jnp.float32)
    return pl.kernel(
        body,
        out_type=jax.ShapeDtypeStruct((2 * NPAD, H), jnp.float32),
        mesh=plsc.VectorSubcoreMesh(**_MESH),
        scratch_types=(
            pltpu.VMEM((NCHUNK, KC), jnp.int32),
            mbuf, mbuf,
            pltpu.SemaphoreType.DMA,
            pltpu.SemaphoreType.DMA,
            pltpu.SemaphoreType.DMA,
            pltpu.SemaphoreType.DMA,
            pltpu.VMEM_SHARED((NPAD, H), jnp.float32),
        ),
        name="sc_scatter_add",
    )(msg, idxd, zeros)


def _ln(y, g, beta):
    mu = jnp.mean(y, axis=-1, keepdims=True)
    var = jnp.mean((y - mu) ** 2, axis=-1, keepdims=True)
    return (y - mu) * lax.rsqrt(var + 1e-5) * g + beta


def _dot(a, b):
    return jnp.dot(a, b, preferred_element_type=jnp.float32)


def _b16r(x):
    u = lax.bitcast_convert_type(x, jnp.int32)
    return (u + 32767 + ((u >> 16) & 1)) >> 16


def _pack2(e, o):
    return (_b16r(e) & 65535) | (_b16r(o) << 16)


def _upk_e(w):
    return lax.bitcast_convert_type(w << 16, jnp.float32)


def _upk_o(w):
    return lax.bitcast_convert_type(w & -65536, jnp.float32)


def _nspec(w=H):
    return pl.BlockSpec((NBLK, w), lambda i: (i, 0))


def _espec(w=H):
    return pl.BlockSpec((EBLK, w), lambda i: (i, 0))


def _wspec(r, c):
    return pl.BlockSpec((r, c), lambda i: (0, 0))


def _encode_nodes(x, w1, b1, w2, b2, g, beta, w1ae, w1ao, w1be, w1bo):
    def body(x_ref, w1_ref, b1_ref, w2_ref, b2_ref, g_ref, be_ref,
             ae_ref, ao_ref, be2_ref, bo_ref, xh_ref, c_ref):
        h = jnp.maximum(_dot(x_ref[...], w1_ref[...]) + b1_ref[...], 0.0)
        y = _dot(h, w2_ref[...]) + b2_ref[...]
        xh = _ln(y, g_ref[...], be_ref[...])
        xh_ref[...] = xh
        pa = _pack2(_dot(xh, ae_ref[...]), _dot(xh, ao_ref[...]))
        pb = _pack2(_dot(xh, be2_ref[...]), _dot(xh, bo_ref[...]))
        c_ref[...] = jnp.concatenate([pa, pb], axis=1)

    return pl.pallas_call(
        body,
        grid=(NPAD // NBLK,),
        in_specs=[_nspec(16), _wspec(16, H), _wspec(1, H), _wspec(H, H),
                  _wspec(1, H), _wspec(1, H), _wspec(1, H),
                  _wspec(H, HW), _wspec(H, HW), _wspec(H, HW), _wspec(H, HW)],
        out_specs=[_nspec(H), _nspec(H)],
        out_shape=[jax.ShapeDtypeStruct((NPAD, H), jnp.float32),
                   jax.ShapeDtypeStruct((NPAD, H), jnp.int32)],
    )(x, w1, b1, w2, b2, g, beta, w1ae, w1ao, w1be, w1bo)


def _edge_mlp_packed(gg, eh_v, w1ce, w1co, b1e, b1o, w2e, w2o, b2, g, beta):
    e1e = _dot(eh_v, w1ce) + b1e
    e1o = _dot(eh_v, w1co) + b1o
    g1 = gg[:, :HW]
    g2 = gg[:, HW:2 * HW]
    h1e = jnp.maximum(_upk_e(g1) + e1e, 0.0)
    h1o = jnp.maximum(_upk_o(g1) + e1o, 0.0)
    y1 = _dot(h1e, w2e) + _dot(h1o, w2o) + b2
    h2e = jnp.maximum(_upk_e(g2) + e1e, 0.0)
    h2o = jnp.maximum(_upk_o(g2) + e1o, 0.0)
    y2 = _dot(h2e, w2e) + _dot(h2o, w2o) + b2
    return _ln(y1, g, beta), _ln(y2, g, beta)


def _edge_step0(g32, we1, be1, we2, be2, ge_, bee,
                w1ce, w1co, b1e, b1o, w2e, w2o, b2, g, beta):
    def body(g_ref, we1_ref, be1_ref, we2_ref, be2_ref, ge_ref, bee_ref,
             w1ce_ref, w1co_ref, b1e_ref, b1o_ref, w2e_ref, w2o_ref, b2_ref,
             g2_ref, blt_ref, msg_ref, enew_ref):
        gg = g_ref[...]
        d0 = _upk_e(gg[:, 128:129])
        d1 = _upk_o(gg[:, 128:129])
        d2 = _upk_e(gg[:, 129:130])
        d3 = _upk_o(gg[:, 129:130])
        d4 = _upk_e(gg[:, 130:131])
        dist = jnp.sqrt(d0 * d0 + d1 * d1)
        dw = jnp.sqrt(d2 * d2 + d3 * d3)
        lane = lax.broadcasted_iota(jnp.int32, (gg.shape[0], 16), 1)
        feat = jnp.where(lane == 0, d0, 0.0)
        feat = jnp.where(lane == 1, d1, feat)
        feat = jnp.where(lane == 2, d2, feat)
        feat = jnp.where(lane == 3, d3, feat)
        feat = jnp.where(lane == 4, d4, feat)
        feat = jnp.where(lane == 5, dist, feat)
        feat = jnp.where(lane == 6, dw, feat)
        h = jnp.maximum(_dot(feat, we1_ref[...]) + be1_ref[...], 0.0)
        y = _dot(h, we2_ref[...]) + be2_ref[...]
        eh_v = _ln(y, ge_ref[...], bee_ref[...])
        msg, de = _edge_mlp_packed(
            gg, eh_v, w1ce_ref[...], w1co_ref[...], b1e_ref[...], b1o_ref[...],
            w2e_ref[...], w2o_ref[...], b2_ref[...], g2_ref[...], blt_ref[...])
        msg_ref[...] = msg
        enew_ref[...] = eh_v + de

    return pl.pallas_call(
        body,
        grid=(E // EBLK,),
        in_specs=[_espec(144), _wspec(16, H), _wspec(1, H),
                  _wspec(H, H), _wspec(1, H), _wspec(1, H), _wspec(1, H),
                  _wspec(H, HW), _wspec(H, HW), _wspec(1, HW), _wspec(1, HW),
                  _wspec(HW, H), _wspec(HW, H), _wspec(1, H),
                  _wspec(1, H), _wspec(1, H)],
        out_specs=[_espec(H), _espec(H)],
        out_shape=[jax.ShapeDtypeStruct((E, H), jnp.float32),
                   jax.ShapeDtypeStruct((E, H), jnp.float32)],
    )(g32, we1, be1, we2, be2, ge_, bee,
      w1ce, w1co, b1e, b1o, w2e, w2o, b2, g, beta)


def _edge_step(eh, g32, w1ce, w1co, b1e, b1o, w2e, w2o, b2, g, beta):
    def body(eh_ref, g_ref, w1ce_ref, w1co_ref, b1e_ref, b1o_ref,
             w2e_ref, w2o_ref, b2_ref, g2_ref, blt_ref, msg_ref, enew_ref):
        eh_v = eh_ref[...]
        msg, de = _edge_mlp_packed(
            g_ref[...], eh_v, w1ce_ref[...], w1co_ref[...], b1e_ref[...],
            b1o_ref[...], w2e_ref[...], w2o_ref[...], b2_ref[...],
            g2_ref[...], blt_ref[...])
        msg_ref[...] = msg
        enew_ref[...] = eh_v + de

    return pl.pallas_call(
        body,
        grid=(E // EBLK,),
        in_specs=[_espec(H), _espec(H),
                  _wspec(H, HW), _wspec(H, HW), _wspec(1, HW), _wspec(1, HW),
                  _wspec(HW, H), _wspec(HW, H), _wspec(1, H),
                  _wspec(1, H), _wspec(1, H)],
        out_specs=[_espec(H), _espec(H)],
        out_shape=[jax.ShapeDtypeStruct((E, H), jnp.float32),
                   jax.ShapeDtypeStruct((E, H), jnp.float32)],
    )(eh, g32, w1ce, w1co, b1e, b1o, w2e, w2o, b2, g, beta)


def _node_step(aggr2, xh, w1na, w1nb, b1, w2, b2, g, beta,
               w1ae, w1ao, w1be, w1bo):
    def body(a_ref, xh_ref, w1na_ref, w1nb_ref, b1_ref, w2_ref, b2_ref,
             g_ref, be_ref, ae_ref, ao_ref, be2_ref, bo_ref, xn_ref, c_ref):
        aggr = a_ref[0] + a_ref[1]
        xh_v = xh_ref[...]
        pre = _dot(aggr, w1na_ref[...]) + _dot(xh_v, w1nb_ref[...]) + b1_ref[...]
        h = jnp.maximum(pre, 0.0)
        y = _dot(h, w2_ref[...]) + b2_ref[...]
        xn = xh_v + _ln(y, g_ref[...], be_ref[...])
        xn_ref[...] = xn
        pa = _pack2(_dot(xn, ae_ref[...]), _dot(xn, ao_ref[...]))
        pb = _pack2(_dot(xn, be2_ref[...]), _dot(xn, bo_ref[...]))
        c_ref[...] = jnp.concatenate([pa, pb], axis=1)

    aspec = pl.BlockSpec((2, NBLK, H), lambda i: (0, i, 0))
    return pl.pallas_call(
        body,
        grid=(NPAD // NBLK,),
        in_specs=[aspec, _nspec(H), _wspec(H, H), _wspec(H, H), _wspec(1, H),
                  _wspec(H, H), _wspec(1, H), _wspec(1, H), _wspec(1, H),
                  _wspec(H, HW), _wspec(H, HW), _wspec(H, HW), _wspec(H, HW)],
        out_specs=[_nspec(H), _nspec(H)],
        out_shape=[jax.ShapeDtypeStruct((NPAD, H), jnp.float32),
                   jax.ShapeDtypeStruct((NPAD, H), jnp.int32)],
    )(aggr2, xh, w1na, w1nb, b1, w2, b2, g, beta, w1ae, w1ao, w1be, w1bo)


def _node_step_last(aggr2, xh, w1na, w1nb, b1, w2, b2, g, beta,
                    wd1a, bd1a, wd2a, wd1b, bd1b, wd2b, bdec):
    def body(a_ref, xh_ref, w1na_ref, w1nb_ref, b1_ref, w2_ref, b2_ref,
             g_ref, be_ref, wd1a_ref, bd1a_ref, wd2a_ref, wd1b_ref, bd1b_ref,
             wd2b_ref, bdec_ref, out_ref):
        aggr = a_ref[0] + a_ref[1]
        xh_v = xh_ref[...]
        pre = _dot(aggr, w1na_ref[...]) + _dot(xh_v, w1nb_ref[...]) + b1_ref[...]
        h = jnp.maximum(pre, 0.0)
        y = _dot(h, w2_ref[...]) + b2_ref[...]
        xn = xh_v + _ln(y, g_ref[...], be_ref[...])
        h1 = jnp.maximum(_dot(xn, wd1a_ref[...]) + bd1a_ref[...], 0.0)
        h2 = jnp.maximum(_dot(xn, wd1b_ref[...]) + bd1b_ref[...], 0.0)
        out_ref[...] = _dot(h1, wd2a_ref[...]) + _dot(h2, wd2b_ref[...]) \
            + bdec_ref[...]

    aspec = pl.BlockSpec((2, NBLK, H), lambda i: (0, i, 0))
    return pl.pallas_call(
        body,
        grid=(NPAD // NBLK,),
        in_specs=[aspec, _nspec(H), _wspec(H, H), _wspec(H, H), _wspec(1, H),
                  _wspec(H, H), _wspec(1, H), _wspec(1, H), _wspec(1, H),
                  _wspec(H, H), _wspec(1, H), _wspec(H, H), _wspec(H, H),
                  _wspec(1, H), _wspec(H, H), _wspec(1, H)],
        out_specs=_nspec(H),
        out_shape=jax.ShapeDtypeStruct((NPAD, H), jnp.float32),
    )(aggr2, xh, w1na, w1nb, b1, w2, b2, g, beta,
      wd1a, bd1a, wd2a, wd1b, bd1b, wd2b, bdec)


def _row(v):
    return v.reshape(1, -1)


def _padn(a):
    return jnp.pad(a, ((0, NPAD - N), (0, 0)))


def _pack_jnp(cols):
    u = lax.bitcast_convert_type(cols, jnp.int32)
    r = (u + 32767 + ((u >> 16) & 1)) >> 16
    return (r[:, 0::2] & 65535) | (r[:, 1::2] << 16)


def kernel(world_pos, mesh_pos, prev_world_pos, phi, prev_phi, swelling_phi,
           swelling_phi_rate, swelling_phi_rate_prev, node_type, mat_param,
           edge_index, params):
    f32 = jnp.float32
    src = edge_index[0].astype(jnp.int32).reshape(NW, NCHUNK, KC)
    dst = edge_index[1].astype(jnp.int32).reshape(NW, NCHUNK, KC)

    x = jnp.concatenate(
        [world_pos - prev_world_pos, phi - prev_phi, swelling_phi,
         swelling_phi_rate, swelling_phi_rate_prev, node_type], axis=-1)
    x = _padn(jnp.pad(x, ((0, 0), (0, 6)))).astype(f32)

    p32 = _pack_jnp(_padn(jnp.pad(
        jnp.concatenate([mesh_pos, world_pos, phi], axis=-1),
        ((0, 0), (0, 27)))).astype(f32))

    ne = params["node_encoder"]
    ee = params["edge_encoder"]
    proc = params["proc"]

    wn1 = jnp.pad(ne["W1"], ((0, 6), (0, 0)))
    we1 = jnp.pad(ee["W1"][jnp.array([0, 1, 3, 4, 6, 2, 5]), :],
                  ((0, 9), (0, 0)))

    ew = [p["edge_mlp"] for p in proc]
    nw_ = [p["node_mlp"] for p in proc]
    w1ae = [w["W1"][:H, 0::2] for w in ew]
    w1ao = [w["W1"][:H, 1::2] for w in ew]
    w1be = [w["W1"][H:2 * H, 0::2] for w in ew]
    w1bo = [w["W1"][H:2 * H, 1::2] for w in ew]
    w1ce = [w["W1"][2 * H:, 0::2] for w in ew]
    w1co = [w["W1"][2 * H:, 1::2] for w in ew]
    b1e = [_row(w["b1"][0::2]) for w in ew]
    b1o = [_row(w["b1"][1::2]) for w in ew]
    w2e = [w["W2"][0::2, :] for w in ew]
    w2o = [w["W2"][1::2, :] for w in ew]

    xh, c = _encode_nodes(x, wn1, _row(ne["b1"]), ne["W2"], _row(ne["b2"]),
                          _row(ne["g"]), _row(ne["beta"]),
                          w1ae[0], w1ao[0], w1be[0], w1bo[0])

    zeros = jnp.zeros((NPAD, H), f32)
    wd = params["world_pos_decoder"]
    pdx = params["phi_decoder"]
    wd2a = jnp.pad(wd["W2"], ((0, 0), (0, H - 2)))
    wd2b = jnp.pad(pdx["W2"], ((0, 0), (2, H - 3)))
    bdec = _row(jnp.pad(jnp.concatenate([wd["b2"], pdx["b2"]]), (0, H - 3)))

    for i in range(3):
        e = ew[i]
        nm = nw_[i]
        if i == 0:
            t0 = jnp.concatenate(
                [c, p32, jnp.zeros((NPAD, 112), jnp.int32)], axis=1)
            g32 = _gather_combine(t0, dst, src, 144, True)
            msg, eh = _edge_step0(
                g32, we1, _row(ee["b1"]), ee["W2"], _row(ee["b2"]),
                _row(ee["g"]), _row(ee["beta"]),
                w1ce[0], w1co[0], b1e[0], b1o[0], w2e[0], w2o[0],
                _row(e["b2"]), _row(e["g"]), _row(e["beta"]))
        else:
            g32 = _gather_combine(c, dst, src, H, False)
            msg, eh = _edge_step(eh, g32, w1ce[i], w1co[i], b1e[i], b1o[i],
                                 w2e[i], w2o[i], _row(e["b2"]),
                                 _row(e["g"]), _row(e["beta"]))
        aggr2 = _scatter_add(msg, dst, zeros).reshape(2, NPAD, H)
        if i < 2:
            xh, c = _node_step(aggr2, xh, nm["W1"][:H], nm["W1"][H:],
                               _row(nm["b1"]), nm["W2"], _row(nm["b2"]),
                               _row(nm["g"]), _row(nm["beta"]),
                               w1ae[i + 1], w1ao[i + 1],
                               w1be[i + 1], w1bo[i + 1])
        else:
            out = _node_step_last(aggr2, xh, nm["W1"][:H], nm["W1"][H:],
                                  _row(nm["b1"]), nm["W2"], _row(nm["b2"]),
                                  _row(nm["g"]), _row(nm["beta"]),
                                  wd["W1"], _row(wd["b1"]), wd2a,
                                  pdx["W1"], _row(pdx["b1"]), wd2b, bdec)
    return out[:N, :3]


# speedup vs baseline: 1.1601x; 1.1601x over previous
import functools

import jax
import jax.numpy as jnp
from jax import lax
from jax.experimental import pallas as pl
from jax.experimental.pallas import tpu as pltpu
from jax.experimental.pallas import tpu_sc as plsc

N = 10000
NPAD = 10240
E = 320000
H = 128
HW = H // 2
NW = 32
EPW = E // NW
KC = 40
NCHUNK = EPW // KC
NBLK = 2048
EBLK = 2560

_MESH = dict(core_axis_name="c", subcore_axis_name="s")


def _gather_combine(table, idxa, idxb, wo, diff):
    ngrp = wo // 16
    assert ngrp == 8 + (1 if diff else 0)

    def body(table_ref, ia_ref, ib_ref, out_ref,
             ia_v, ib_v, a0, b0, o0, a1, b1, o1, sg0, sg1, ss0, ss1):
        cid = lax.axis_index("c")
        sid = lax.axis_index("s")
        wid = sid * 2 + cid
        pltpu.sync_copy(ia_ref.at[wid], ia_v)
        pltpu.sync_copy(ib_ref.at[wid], ib_v)
        bufs = ((a0, b0, o0, sg0, ss0), (a1, b1, o1, sg1, ss1))

        def issue(j, p):
            a, b, _, sg, _ = bufs[p]
            pltpu.async_copy(table_ref.at[ia_v.at[j]], a, sg)
            pltpu.async_copy(table_ref.at[ib_v.at[j]], b, sg)

        def wait_gather(p):
            a, b, sg = bufs[p][0], bufs[p][1], bufs[p][3]
            pltpu.make_async_copy(table_ref.at[ia_v.at[0]], a, sg).wait()
            pltpu.make_async_copy(table_ref.at[ib_v.at[0]], b, sg).wait()

        def wait_store(p):
            o, ss = bufs[p][2], bufs[p][4]
            pltpu.make_async_copy(o, out_ref.at[pl.ds(0, KC)], ss).wait()

        def combine_store(j, p):
            a, b, o, _, ss = bufs[p]

            def fb(w):
                return lax.bitcast_convert_type(w, jnp.float32)

            def ib(x):
                return lax.bitcast_convert_type(x, jnp.int32)

            def rnd(u):
                return u + 32767 + ((u >> 16) & 1)

            def comb(wa, wb, sign):
                ae = fb(wa << 16)
                ao = fb(wa & -65536)
                be_ = fb(wb << 16)
                bo = fb(wb & -65536)
                if sign < 0:
                    se, so = be_ - ae, bo - ao
                else:
                    se, so = ae + be_, ao + bo
                re = lax.shift_right_logical(rnd(ib(se)), 16)
                ro = rnd(ib(so)) & -65536
                return re | ro

            def row(r, carry):
                for gr in range(ngrp):
                    sl = pl.ds(gr * 16, 16)
                    if gr < 4:
                        v = comb(a[r, sl], b[r, pl.ds(gr * 16 + 64, 16)], 1)
                    elif gr < 8:
                        v = comb(a[r, sl], b[r, pl.ds(gr * 16 - 64, 16)], 1)
                    else:
                        v = comb(a[r, sl], b[r, sl], -1)
                    o[r, sl] = v
                return carry

            lax.fori_loop(0, KC, row, 0)
            pltpu.async_copy(o, out_ref.at[pl.ds(wid * EPW + j * KC, KC)], ss)

        issue(0, 0)

        def step(j2, carry):
            issue(2 * j2 + 1, 1)
            wait_gather(0)

            @pl.when(j2 > 0)
            def _():
                wait_store(0)

            combine_store(2 * j2, 0)

            @pl.when(j2 + 1 < NCHUNK // 2)
            def _():
                issue(2 * j2 + 2, 0)

            wait_gather(1)

            @pl.when(j2 > 0)
            def _():
                wait_store(1)

            combine_store(2 * j2 + 1, 1)
            return carry

        lax.fori_loop(0, NCHUNK // 2, step, 0)
        wait_store(0)
        wait_store(1)

    wt = table.shape[1]
    gbuf = pltpu.VMEM((KC, wt), jnp.int32)
    obuf = pltpu.VMEM((KC, wo), jnp.int32)
    return pl.kernel(
        body,
        out_type=jax.ShapeDtypeStruct((E, wo), jnp.int32),
        mesh=plsc.VectorSubcoreMesh(**_MESH),
        scratch_types=(
            pltpu.VMEM((NCHUNK, KC), jnp.int32),
            pltpu.VMEM((NCHUNK, KC), jnp.int32),
            gbuf, gbuf, obuf, gbuf, gbuf, obuf,
            pltpu.SemaphoreType.DMA,
            pltpu.SemaphoreType.DMA,
            pltpu.SemaphoreType.DMA,
            pltpu.SemaphoreType.DMA,
        ),
        name=f"sc_gather_combine_{wt}_{wo}",
    )(table, idxa, idxb)


def _scatter_add(msg, idxd, zeros):
    def body(msg_ref, idx_ref, zeros_ref, out_ref,
             idx_v, m0, m1, sl0, sl1, ss0, ss1, aggr_sh):
        cid = lax.axis_index("c")
        sid = lax.axis_index("s")
        wid = sid * 2 + cid

        @pl.when(sid == 0)
        def _():
            pltpu.sync_copy(zeros_ref, aggr_sh)

        plsc.subcore_barrier()
        pltpu.sync_copy(idx_ref.at[wid], idx_v)
        bufs = ((m0, sl0, ss0), (m1, sl1, ss1))

        def load(j, p):
            m, sl, _ = bufs[p]
            pltpu.async_copy(msg_ref.at[pl.ds(wid * EPW + j * KC, KC)], m, sl)

        def wait_load(p):
            m, sl, _ = bufs[p]
            pltpu.make_async_copy(msg_ref.at[pl.ds(0, KC)], m, sl).wait()

        def scat(j, p):
            m, _, ss = bufs[p]
            pltpu.async_copy(m, aggr_sh.at[idx_v.at[j]], ss, add=True)

        def wait_scat(p):
            m, _, ss = bufs[p]
            pltpu.make_async_copy(m, aggr_sh.at[idx_v.at[0]], ss).wait()

        load(0, 0)

        def step(j2, carry):
            load(2 * j2 + 1, 1)
            wait_load(0)
            scat(2 * j2, 0)
            wait_scat(0)

            @pl.when(j2 + 1 < NCHUNK // 2)
            def _():
                load(2 * j2 + 2, 0)

            wait_load(1)
            scat(2 * j2 + 1, 1)
            wait_scat(1)
            return carry

        lax.fori_loop(0, NCHUNK // 2, step, 0)
        plsc.subcore_barrier()
        rows = NPAD // 16
        pltpu.sync_copy(aggr_sh.at[pl.ds(sid * rows, rows)],
                        out_ref.at[pl.ds(cid * NPAD + sid * rows, rows)])

    mbuf = pltpu.VMEM((KC, H), jnp.float32)
    return pl.kernel(
        body,
        out_type=jax.ShapeDtypeStruct((2 * NPAD, H), jnp.float32),
        mesh=plsc.VectorSubcoreMesh(**_MESH),
        scratch_types=(
            pltpu.VMEM((NCHUNK, KC), jnp.int32),
            mbuf, mbuf,
            pltpu.SemaphoreType.DMA,
            pltpu.SemaphoreType.DMA,
            pltpu.SemaphoreType.DMA,
            pltpu.SemaphoreType.DMA,
            pltpu.VMEM_SHARED((NPAD, H), jnp.float32),
        ),
        name="sc_scatter_add",
    )(msg, idxd, zeros)


def _ln(y, g, beta):
    mu = jnp.mean(y, axis=-1, keepdims=True)
    var = jnp.mean((y - mu) ** 2, axis=-1, keepdims=True)
    return (y - mu) * lax.rsqrt(var + 1e-5) * g + beta


def _dot(a, b):
    return jnp.dot(a, b, preferred_element_type=jnp.float32)


def _b16r(x):
    u = lax.bitcast_convert_type(x, jnp.int32)
    return (u + 32767 + ((u >> 16) & 1)) >> 16


def _pack2(e, o):
    return (_b16r(e) & 65535) | (_b16r(o) << 16)


def _upk_e(w):
    return lax.bitcast_convert_type(w << 16, jnp.float32)


def _upk_o(w):
    return lax.bitcast_convert_type(w & -65536, jnp.float32)


def _nspec(w=H):
    return pl.BlockSpec((NBLK, w), lambda i: (i, 0))


def _espec(w=H):
    return pl.BlockSpec((EBLK, w), lambda i: (i, 0))


def _wspec(r, c):
    return pl.BlockSpec((r, c), lambda i: (0, 0))


def _encode_nodes(x, w1, b1, w2, b2, g, beta, w1a, w1b):
    def body(x_ref, w1_ref, b1_ref, w2_ref, b2_ref, g_ref, be_ref,
             wa_ref, wb_ref, xh_ref, c_ref):
        h = jnp.maximum(_dot(x_ref[...], w1_ref[...]) + b1_ref[...], 0.0)
        y = _dot(h, w2_ref[...]) + b2_ref[...]
        xh = _ln(y, g_ref[...], be_ref[...])
        xh_ref[...] = xh
        ca = _dot(xh, wa_ref[...])
        cb = _dot(xh, wb_ref[...])
        pa = _pack2(ca[:, :HW], ca[:, HW:])
        pb = _pack2(cb[:, :HW], cb[:, HW:])
        c_ref[...] = jnp.concatenate([pa, pb], axis=1)

    return pl.pallas_call(
        body,
        grid=(NPAD // NBLK,),
        in_specs=[_nspec(16), _wspec(16, H), _wspec(1, H), _wspec(H, H),
                  _wspec(1, H), _wspec(1, H), _wspec(1, H),
                  _wspec(H, H), _wspec(H, H)],
        out_specs=[_nspec(H), _nspec(H)],
        out_shape=[jax.ShapeDtypeStruct((NPAD, H), jnp.float32),
                   jax.ShapeDtypeStruct((NPAD, H), jnp.int32)],
    )(x, w1, b1, w2, b2, g, beta, w1a, w1b)


def _edge_mlp_packed(gg, eh_v, w1c, b1, w2, b2, g, beta):
    e1 = _dot(eh_v, w1c) + b1
    ge = _upk_e(gg[:, :2 * HW])
    go = _upk_o(gg[:, :2 * HW])
    g1 = jnp.concatenate([ge[:, :HW], go[:, :HW]], axis=1)
    g2 = jnp.concatenate([ge[:, HW:], go[:, HW:]], axis=1)
    h1 = jnp.maximum(g1 + e1, 0.0)
    y1 = _dot(h1, w2) + b2
    h2 = jnp.maximum(g2 + e1, 0.0)
    y2 = _dot(h2, w2) + b2
    return _ln(y1, g, beta), _ln(y2, g, beta)


def _edge_step0(g32, we1, be1, we2, be2, ge_, bee, w1c, b1, w2, b2, g, beta):
    def body(g_ref, we1_ref, be1_ref, we2_ref, be2_ref, ge_ref, bee_ref,
             w1c_ref, b1_ref, w2_ref, b2_ref, g2_ref, blt_ref,
             msg_ref, enew_ref):
        gg = g_ref[...]
        d = _upk_e(gg[:, 128:144])
        dist = jnp.sqrt(d[:, 0:1] ** 2 + d[:, 1:2] ** 2)
        dw = jnp.sqrt(d[:, 2:3] ** 2 + d[:, 3:4] ** 2)
        lane = lax.broadcasted_iota(jnp.int32, d.shape, 1)
        feat = jnp.where(lane == 5, dist, d)
        feat = jnp.where(lane == 6, dw, feat)
        h = jnp.maximum(_dot(feat, we1_ref[...]) + be1_ref[...], 0.0)
        y = _dot(h, we2_ref[...]) + be2_ref[...]
        eh_v = _ln(y, ge_ref[...], bee_ref[...])
        msg, de = _edge_mlp_packed(
            gg, eh_v, w1c_ref[...], b1_ref[...], w2_ref[...], b2_ref[...],
            g2_ref[...], blt_ref[...])
        msg_ref[...] = msg
        enew_ref[...] = eh_v + de

    return pl.pallas_call(
        body,
        grid=(E // EBLK,),
        in_specs=[_espec(144), _wspec(16, H), _wspec(1, H),
                  _wspec(H, H), _wspec(1, H), _wspec(1, H), _wspec(1, H),
                  _wspec(H, H), _wspec(1, H), _wspec(H, H), _wspec(1, H),
                  _wspec(1, H), _wspec(1, H)],
        out_specs=[_espec(H), _espec(H)],
        out_shape=[jax.ShapeDtypeStruct((E, H), jnp.float32),
                   jax.ShapeDtypeStruct((E, H), jnp.float32)],
    )(g32, we1, be1, we2, be2, ge_, bee, w1c, b1, w2, b2, g, beta)


def _edge_step(eh, g32, w1c, b1, w2, b2, g, beta):
    def body(eh_ref, g_ref, w1c_ref, b1_ref, w2_ref, b2_ref,
             g2_ref, blt_ref, msg_ref, enew_ref):
        eh_v = eh_ref[...]
        msg, de = _edge_mlp_packed(
            g_ref[...], eh_v, w1c_ref[...], b1_ref[...], w2_ref[...],
            b2_ref[...], g2_ref[...], blt_ref[...])
        msg_ref[...] = msg
        enew_ref[...] = eh_v + de

    return pl.pallas_call(
        body,
        grid=(E // EBLK,),
        in_specs=[_espec(H), _espec(H),
                  _wspec(H, H), _wspec(1, H), _wspec(H, H), _wspec(1, H),
                  _wspec(1, H), _wspec(1, H)],
        out_specs=[_espec(H), _espec(H)],
        out_shape=[jax.ShapeDtypeStruct((E, H), jnp.float32),
                   jax.ShapeDtypeStruct((E, H), jnp.float32)],
    )(eh, g32, w1c, b1, w2, b2, g, beta)


def _node_step(aggr2, xh, w1na, w1nb, b1, w2, b2, g, beta, w1a, w1b):
    def body(a_ref, xh_ref, w1na_ref, w1nb_ref, b1_ref, w2_ref, b2_ref,
             g_ref, be_ref, wa_ref, wb_ref, xn_ref, c_ref):
        aggr = a_ref[0] + a_ref[1]
        xh_v = xh_ref[...]
        pre = _dot(aggr, w1na_ref[...]) + _dot(xh_v, w1nb_ref[...]) + b1_ref[...]
        h = jnp.maximum(pre, 0.0)
        y = _dot(h, w2_ref[...]) + b2_ref[...]
        xn = xh_v + _ln(y, g_ref[...], be_ref[...])
        xn_ref[...] = xn
        ca = _dot(xn, wa_ref[...])
        cb = _dot(xn, wb_ref[...])
        pa = _pack2(ca[:, :HW], ca[:, HW:])
        pb = _pack2(cb[:, :HW], cb[:, HW:])
        c_ref[...] = jnp.concatenate([pa, pb], axis=1)

    aspec = pl.BlockSpec((2, NBLK, H), lambda i: (0, i, 0))
    return pl.pallas_call(
        body,
        grid=(NPAD // NBLK,),
        in_specs=[aspec, _nspec(H), _wspec(H, H), _wspec(H, H), _wspec(1, H),
                  _wspec(H, H), _wspec(1, H), _wspec(1, H), _wspec(1, H),
                  _wspec(H, H), _wspec(H, H)],
        out_specs=[_nspec(H), _nspec(H)],
        out_shape=[jax.ShapeDtypeStruct((NPAD, H), jnp.float32),
                   jax.ShapeDtypeStruct((NPAD, H), jnp.int32)],
    )(aggr2, xh, w1na, w1nb, b1, w2, b2, g, beta, w1a, w1b)


def _node_step_last(aggr2, xh, w1na, w1nb, b1, w2, b2, g, beta,
                    wd1a, bd1a, wd2a, wd1b, bd1b, wd2b, bdec):
    def body(a_ref, xh_ref, w1na_ref, w1nb_ref, b1_ref, w2_ref, b2_ref,
             g_ref, be_ref, wd1a_ref, bd1a_ref, wd2a_ref, wd1b_ref, bd1b_ref,
             wd2b_ref, bdec_ref, out_ref):
        aggr = a_ref[0] + a_ref[1]
        xh_v = xh_ref[...]
        pre = _dot(aggr, w1na_ref[...]) + _dot(xh_v, w1nb_ref[...]) + b1_ref[...]
        h = jnp.maximum(pre, 0.0)
        y = _dot(h, w2_ref[...]) + b2_ref[...]
        xn = xh_v + _ln(y, g_ref[...], be_ref[...])
        h1 = jnp.maximum(_dot(xn, wd1a_ref[...]) + bd1a_ref[...], 0.0)
        h2 = jnp.maximum(_dot(xn, wd1b_ref[...]) + bd1b_ref[...], 0.0)
        out_ref[...] = _dot(h1, wd2a_ref[...]) + _dot(h2, wd2b_ref[...]) \
            + bdec_ref[...]

    aspec = pl.BlockSpec((2, NBLK, H), lambda i: (0, i, 0))
    return pl.pallas_call(
        body,
        grid=(NPAD // NBLK,),
        in_specs=[aspec, _nspec(H), _wspec(H, H), _wspec(H, H), _wspec(1, H),
                  _wspec(H, H), _wspec(1, H), _wspec(1, H), _wspec(1, H),
                  _wspec(H, H), _wspec(1, H), _wspec(H, H), _wspec(H, H),
                  _wspec(1, H), _wspec(H, H), _wspec(1, H)],
        out_specs=_nspec(H),
        out_shape=jax.ShapeDtypeStruct((NPAD, H), jnp.float32),
    )(aggr2, xh, w1na, w1nb, b1, w2, b2, g, beta,
      wd1a, bd1a, wd2a, wd1b, bd1b, wd2b, bdec)


def _row(v):
    return v.reshape(1, -1)


def _padn(a):
    return jnp.pad(a, ((0, NPAD - N), (0, 0)))


def _pack_jnp(cols):
    k = cols.shape[1] // 2
    u = lax.bitcast_convert_type(cols, jnp.int32)
    r = (u + 32767 + ((u >> 16) & 1)) >> 16
    return (r[:, :k] & 65535) | (r[:, k:] << 16)


def kernel(world_pos, mesh_pos, prev_world_pos, phi, prev_phi, swelling_phi,
           swelling_phi_rate, swelling_phi_rate_prev, node_type, mat_param,
           edge_index, params):
    f32 = jnp.float32
    src = edge_index[0].astype(jnp.int32).reshape(NW, NCHUNK, KC)
    dst = edge_index[1].astype(jnp.int32).reshape(NW, NCHUNK, KC)

    x = jnp.concatenate(
        [world_pos - prev_world_pos, phi - prev_phi, swelling_phi,
         swelling_phi_rate, swelling_phi_rate_prev, node_type], axis=-1)
    x = _padn(jnp.pad(x, ((0, 0), (0, 6)))).astype(f32)

    p32 = _pack_jnp(_padn(jnp.pad(
        jnp.concatenate([mesh_pos, world_pos, phi], axis=-1),
        ((0, 0), (0, 27)))).astype(f32))

    ne = params["node_encoder"]
    ee = params["edge_encoder"]
    proc = params["proc"]

    wn1 = jnp.pad(ne["W1"], ((0, 6), (0, 0)))
    we1 = jnp.pad(ee["W1"][jnp.array([0, 1, 3, 4, 6, 2, 5]), :],
                  ((0, 9), (0, 0)))

    ew = [p["edge_mlp"] for p in proc]
    nw_ = [p["node_mlp"] for p in proc]
    w1a = [w["W1"][:H] for w in ew]
    w1b = [w["W1"][H:2 * H] for w in ew]
    w1c = [w["W1"][2 * H:] for w in ew]

    xh, c = _encode_nodes(x, wn1, _row(ne["b1"]), ne["W2"], _row(ne["b2"]),
                          _row(ne["g"]), _row(ne["beta"]), w1a[0], w1b[0])

    zeros = jnp.zeros((NPAD, H), f32)
    wd = params["world_pos_decoder"]
    pdx = params["phi_decoder"]
    wd2a = jnp.pad(wd["W2"], ((0, 0), (0, H - 2)))
    wd2b = jnp.pad(pdx["W2"], ((0, 0), (2, H - 3)))
    bdec = _row(jnp.pad(jnp.concatenate([wd["b2"], pdx["b2"]]), (0, H - 3)))

    for i in range(3):
        e = ew[i]
        nm = nw_[i]
        if i == 0:
            t0 = jnp.concatenate(
                [c, p32, jnp.zeros((NPAD, 112), jnp.int32)], axis=1)
            g32 = _gather_combine(t0, dst, src, 144, True)
            msg, eh = _edge_step0(
                g32, we1, _row(ee["b1"]), ee["W2"], _row(ee["b2"]),
                _row(ee["g"]), _row(ee["beta"]), w1c[0], _row(e["b1"]),
                e["W2"], _row(e["b2"]), _row(e["g"]), _row(e["beta"]))
        else:
            g32 = _gather_combine(c, dst, src, H, False)
            msg, eh = _edge_step(eh, g32, w1c[i], _row(e["b1"]), e["W2"],
                                 _row(e["b2"]), _row(e["g"]), _row(e["beta"]))
        aggr2 = _scatter_add(msg, dst, zeros).reshape(2, NPAD, H)
        if i < 2:
            xh, c = _node_step(aggr2, xh, nm["W1"][:H], nm["W1"][H:],
                               _row(nm["b1"]), nm["W2"], _row(nm["b2"]),
                               _row(nm["g"]), _row(nm["beta"]),
                               w1a[i + 1], w1b[i + 1])
        else:
            out = _node_step_last(aggr2, xh, nm["W1"][:H], nm["W1"][H:],
                                  _row(nm["b1"]), nm["W2"], _row(nm["b2"]),
                                  _row(nm["g"]), _row(nm["beta"]),
                                  wd["W1"], _row(wd["b1"]), wd2a,
                                  pdx["W1"], _row(pdx["b1"]), wd2b, bdec)
    return out[:N, :3]


# speedup vs baseline: 1.1719x; 1.0101x over previous
import functools

import jax
import jax.numpy as jnp
from jax import lax
from jax.experimental import pallas as pl
from jax.experimental.pallas import tpu as pltpu
from jax.experimental.pallas import tpu_sc as plsc

N = 10000
NPAD = 10240
E = 320000
H = 128
HW = H // 2
NW = 32
EPW = E // NW
KC = 40
NCHUNK = EPW // KC
NBLK = 2048
EBLK = 2560

_MESH = dict(core_axis_name="c", subcore_axis_name="s")


def _gather_combine(table, idxa, idxb, wo, diff):
    ngrp = wo // 16
    assert ngrp == 8 + (1 if diff else 0)

    def body(table_ref, ia_ref, ib_ref, out_ref,
             ia_v, ib_v, a0, b0, o0, a1, b1, o1, sg0, sg1, ss0, ss1):
        cid = lax.axis_index("c")
        sid = lax.axis_index("s")
        wid = sid * 2 + cid
        pltpu.sync_copy(ia_ref.at[wid], ia_v)
        pltpu.sync_copy(ib_ref.at[wid], ib_v)
        bufs = ((a0, b0, o0, sg0, ss0), (a1, b1, o1, sg1, ss1))

        def issue(j, p):
            a, b, _, sg, _ = bufs[p]
            pltpu.async_copy(table_ref.at[ia_v.at[j]], a, sg)
            pltpu.async_copy(table_ref.at[ib_v.at[j]], b, sg)

        def wait_gather(p):
            a, b, sg = bufs[p][0], bufs[p][1], bufs[p][3]
            pltpu.make_async_copy(table_ref.at[ia_v.at[0]], a, sg).wait()
            pltpu.make_async_copy(table_ref.at[ib_v.at[0]], b, sg).wait()

        def wait_store(p):
            o, ss = bufs[p][2], bufs[p][4]
            pltpu.make_async_copy(o, out_ref.at[pl.ds(0, KC)], ss).wait()

        def combine_store(j, p):
            a, b, o, _, ss = bufs[p]

            def fb(w):
                return lax.bitcast_convert_type(w, jnp.float32)

            def ib(x):
                return lax.bitcast_convert_type(x, jnp.int32)

            def rnd(u):
                return u + 32767 + ((u >> 16) & 1)

            def comb(wa, wb, sign):
                ae = fb(wa << 16)
                ao = fb(wa & -65536)
                be_ = fb(wb << 16)
                bo = fb(wb & -65536)
                if sign < 0:
                    se, so = be_ - ae, bo - ao
                else:
                    se, so = ae + be_, ao + bo
                re = lax.shift_right_logical(rnd(ib(se)), 16)
                ro = rnd(ib(so)) & -65536
                return re | ro

            def row(r, carry):
                for gr in range(ngrp):
                    sl = pl.ds(gr * 16, 16)
                    if gr < 4:
                        v = comb(a[r, sl], b[r, pl.ds(gr * 16 + 64, 16)], 1)
                    elif gr < 8:
                        v = comb(a[r, sl], b[r, pl.ds(gr * 16 - 64, 16)], 1)
                    else:
                        v = ib(fb(b[r, sl]) - fb(a[r, sl]))
                    o[r, sl] = v
                return carry

            lax.fori_loop(0, KC, row, 0)
            pltpu.async_copy(o, out_ref.at[pl.ds(wid * EPW + j * KC, KC)], ss)

        issue(0, 0)

        def step(j2, carry):
            issue(2 * j2 + 1, 1)
            wait_gather(0)

            @pl.when(j2 > 0)
            def _():
                wait_store(0)

            combine_store(2 * j2, 0)

            @pl.when(j2 + 1 < NCHUNK // 2)
            def _():
                issue(2 * j2 + 2, 0)

            wait_gather(1)

            @pl.when(j2 > 0)
            def _():
                wait_store(1)

            combine_store(2 * j2 + 1, 1)
            return carry

        lax.fori_loop(0, NCHUNK // 2, step, 0)
        wait_store(0)
        wait_store(1)

    wt = table.shape[1]
    gbuf = pltpu.VMEM((KC, wt), jnp.int32)
    obuf = pltpu.VMEM((KC, wo), jnp.int32)
    return pl.kernel(
        body,
        out_type=jax.ShapeDtypeStruct((E, wo), jnp.int32),
        mesh=plsc.VectorSubcoreMesh(**_MESH),
        scratch_types=(
            pltpu.VMEM((NCHUNK, KC), jnp.int32),
            pltpu.VMEM((NCHUNK, KC), jnp.int32),
            gbuf, gbuf, obuf, gbuf, gbuf, obuf,
            pltpu.SemaphoreType.DMA,
            pltpu.SemaphoreType.DMA,
            pltpu.SemaphoreType.DMA,
            pltpu.SemaphoreType.DMA,
        ),
        name=f"sc_gather_combine_{wt}_{wo}",
    )(table, idxa, idxb)


def _scatter_add(msg, idxd, zeros):
    def body(msg_ref, idx_ref, zeros_ref, out_ref,
             idx_v, m0, m1, sl0, sl1, ss0, ss1, aggr_sh):
        cid = lax.axis_index("c")
        sid = lax.axis_index("s")
        wid = sid * 2 + cid

        @pl.when(sid == 0)
        def _():
            pltpu.sync_copy(zeros_ref, aggr_sh)

        plsc.subcore_barrier()
        pltpu.sync_copy(idx_ref.at[wid], idx_v)
        bufs = ((m0, sl0, ss0), (m1, sl1, ss1))

        def load(j, p):
            m, sl, _ = bufs[p]
            pltpu.async_copy(msg_ref.at[pl.ds(wid * EPW + j * KC, KC)], m, sl)

        def wait_load(p):
            m, sl, _ = bufs[p]
            pltpu.make_async_copy(msg_ref.at[pl.ds(0, KC)], m, sl).wait()

        def scat(j, p):
            m, _, ss = bufs[p]
            pltpu.async_copy(m, aggr_sh.at[idx_v.at[j]], ss, add=True)

        def wait_scat(p):
            m, _, ss = bufs[p]
            pltpu.make_async_copy(m, aggr_sh.at[idx_v.at[0]], ss).wait()

        load(0, 0)

        def step(j2, carry):
            load(2 * j2 + 1, 1)
            wait_load(0)
            scat(2 * j2, 0)
            wait_scat(0)

            @pl.when(j2 + 1 < NCHUNK // 2)
            def _():
                load(2 * j2 + 2, 0)

            wait_load(1)
            scat(2 * j2 + 1, 1)
            wait_scat(1)
            return carry

        lax.fori_loop(0, NCHUNK // 2, step, 0)
        plsc.subcore_barrier()
        rows = NPAD // 16
        pltpu.sync_copy(aggr_sh.at[pl.ds(sid * rows, rows)],
                        out_ref.at[pl.ds(cid * NPAD + sid * rows, rows)])

    mbuf = pltpu.VMEM((KC, H), jnp.float32)
    return pl.kernel(
        body,
        out_type=jax.ShapeDtypeStruct((2 * NPAD, H), jnp.float32),
        mesh=plsc.VectorSubcoreMesh(**_MESH),
        scratch_types=(
            pltpu.VMEM((NCHUNK, KC), jnp.int32),
            mbuf, mbuf,
            pltpu.SemaphoreType.DMA,
            pltpu.SemaphoreType.DMA,
            pltpu.SemaphoreType.DMA,
            pltpu.SemaphoreType.DMA,
            pltpu.VMEM_SHARED((NPAD, H), jnp.float32),
        ),
        name="sc_scatter_add",
    )(msg, idxd, zeros)


def _ln(y, g, beta):
    mu = jnp.mean(y, axis=-1, keepdims=True)
    var = jnp.mean((y - mu) ** 2, axis=-1, keepdims=True)
    return (y - mu) * lax.rsqrt(var + 1e-5) * g + beta


def _dot(a, b):
    return jnp.dot(a, b, preferred_element_type=jnp.float32)


def _b16r(x):
    u = lax.bitcast_convert_type(x, jnp.int32)
    return (u + 32767 + ((u >> 16) & 1)) >> 16


def _pack2(e, o):
    return (_b16r(e) & 65535) | (_b16r(o) << 16)


def _upk_e(w):
    return lax.bitcast_convert_type(w << 16, jnp.float32)


def _upk_o(w):
    return lax.bitcast_convert_type(w & -65536, jnp.float32)


def _nspec(w=H):
    return pl.BlockSpec((NBLK, w), lambda i: (i, 0))


def _espec(w=H):
    return pl.BlockSpec((EBLK, w), lambda i: (i, 0))


def _wspec(r, c):
    return pl.BlockSpec((r, c), lambda i: (0, 0))


def _encode_nodes(x, w1, b1, w2, b2, g, beta, w1a, w1b):
    def body(x_ref, w1_ref, b1_ref, w2_ref, b2_ref, g_ref, be_ref,
             wa_ref, wb_ref, xh_ref, c_ref):
        h = jnp.maximum(_dot(x_ref[...], w1_ref[...]) + b1_ref[...], 0.0)
        y = _dot(h, w2_ref[...]) + b2_ref[...]
        xh = _ln(y, g_ref[...], be_ref[...])
        xh_ref[...] = xh
        ca = _dot(xh, wa_ref[...])
        cb = _dot(xh, wb_ref[...])
        pa = _pack2(ca[:, :HW], ca[:, HW:])
        pb = _pack2(cb[:, :HW], cb[:, HW:])
        c_ref[...] = jnp.concatenate([pa, pb], axis=1)

    return pl.pallas_call(
        body,
        grid=(NPAD // NBLK,),
        in_specs=[_nspec(16), _wspec(16, H), _wspec(1, H), _wspec(H, H),
                  _wspec(1, H), _wspec(1, H), _wspec(1, H),
                  _wspec(H, H), _wspec(H, H)],
        out_specs=[_nspec(H), _nspec(H)],
        out_shape=[jax.ShapeDtypeStruct((NPAD, H), jnp.float32),
                   jax.ShapeDtypeStruct((NPAD, H), jnp.int32)],
    )(x, w1, b1, w2, b2, g, beta, w1a, w1b)


def _edge_mlp_packed(gg, eh_v, w1c, b1, w2, b2, g, beta):
    e1 = _dot(eh_v, w1c) + b1
    ge = _upk_e(gg[:, :2 * HW])
    go = _upk_o(gg[:, :2 * HW])
    g1 = jnp.concatenate([ge[:, :HW], go[:, :HW]], axis=1)
    g2 = jnp.concatenate([ge[:, HW:], go[:, HW:]], axis=1)
    h1 = jnp.maximum(g1 + e1, 0.0)
    y1 = _dot(h1, w2) + b2
    h2 = jnp.maximum(g2 + e1, 0.0)
    y2 = _dot(h2, w2) + b2
    return _ln(y1, g, beta), _ln(y2, g, beta)


def _edge_step0(g32, we1, be1, we2, be2, ge_, bee, w1c, b1, w2, b2, g, beta):
    def body(g_ref, we1_ref, be1_ref, we2_ref, be2_ref, ge_ref, bee_ref,
             w1c_ref, b1_ref, w2_ref, b2_ref, g2_ref, blt_ref,
             msg_ref, enew_ref):
        gg = g_ref[...]
        d = lax.bitcast_convert_type(gg[:, 128:144], jnp.float32)
        dist = jnp.sqrt(d[:, 0:1] ** 2 + d[:, 1:2] ** 2)
        dw = jnp.sqrt(d[:, 2:3] ** 2 + d[:, 3:4] ** 2)
        lane = lax.broadcasted_iota(jnp.int32, d.shape, 1)
        feat = jnp.where(lane == 5, dist, d)
        feat = jnp.where(lane == 6, dw, feat)
        h = jnp.maximum(_dot(feat, we1_ref[...]) + be1_ref[...], 0.0)
        y = _dot(h, we2_ref[...]) + be2_ref[...]
        eh_v = _ln(y, ge_ref[...], bee_ref[...])
        msg, de = _edge_mlp_packed(
            gg, eh_v, w1c_ref[...], b1_ref[...], w2_ref[...], b2_ref[...],
            g2_ref[...], blt_ref[...])
        msg_ref[...] = msg
        enew_ref[...] = eh_v + de

    return pl.pallas_call(
        body,
        grid=(E // EBLK,),
        in_specs=[_espec(144), _wspec(16, H), _wspec(1, H),
                  _wspec(H, H), _wspec(1, H), _wspec(1, H), _wspec(1, H),
                  _wspec(H, H), _wspec(1, H), _wspec(H, H), _wspec(1, H),
                  _wspec(1, H), _wspec(1, H)],
        out_specs=[_espec(H), _espec(H)],
        out_shape=[jax.ShapeDtypeStruct((E, H), jnp.float32),
                   jax.ShapeDtypeStruct((E, H), jnp.float32)],
    )(g32, we1, be1, we2, be2, ge_, bee, w1c, b1, w2, b2, g, beta)


def _edge_step(eh, g32, w1c, b1, w2, b2, g, beta):
    def body(eh_ref, g_ref, w1c_ref, b1_ref, w2_ref, b2_ref,
             g2_ref, blt_ref, msg_ref, enew_ref):
        eh_v = eh_ref[...]
        msg, de = _edge_mlp_packed(
            g_ref[...], eh_v, w1c_ref[...], b1_ref[...], w2_ref[...],
            b2_ref[...], g2_ref[...], blt_ref[...])
        msg_ref[...] = msg
        enew_ref[...] = eh_v + de

    return pl.pallas_call(
        body,
        grid=(E // EBLK,),
        in_specs=[_espec(H), _espec(H),
                  _wspec(H, H), _wspec(1, H), _wspec(H, H), _wspec(1, H),
                  _wspec(1, H), _wspec(1, H)],
        out_specs=[_espec(H), _espec(H)],
        out_shape=[jax.ShapeDtypeStruct((E, H), jnp.float32),
                   jax.ShapeDtypeStruct((E, H), jnp.float32)],
    )(eh, g32, w1c, b1, w2, b2, g, beta)


def _node_step(aggr2, xh, w1na, w1nb, b1, w2, b2, g, beta, w1a, w1b):
    def body(a_ref, xh_ref, w1na_ref, w1nb_ref, b1_ref, w2_ref, b2_ref,
             g_ref, be_ref, wa_ref, wb_ref, xn_ref, c_ref):
        aggr = a_ref[0] + a_ref[1]
        xh_v = xh_ref[...]
        pre = _dot(aggr, w1na_ref[...]) + _dot(xh_v, w1nb_ref[...]) + b1_ref[...]
        h = jnp.maximum(pre, 0.0)
        y = _dot(h, w2_ref[...]) + b2_ref[...]
        xn = xh_v + _ln(y, g_ref[...], be_ref[...])
        xn_ref[...] = xn
        ca = _dot(xn, wa_ref[...])
        cb = _dot(xn, wb_ref[...])
        pa = _pack2(ca[:, :HW], ca[:, HW:])
        pb = _pack2(cb[:, :HW], cb[:, HW:])
        c_ref[...] = jnp.concatenate([pa, pb], axis=1)

    aspec = pl.BlockSpec((2, NBLK, H), lambda i: (0, i, 0))
    return pl.pallas_call(
        body,
        grid=(NPAD // NBLK,),
        in_specs=[aspec, _nspec(H), _wspec(H, H), _wspec(H, H), _wspec(1, H),
                  _wspec(H, H), _wspec(1, H), _wspec(1, H), _wspec(1, H),
                  _wspec(H, H), _wspec(H, H)],
        out_specs=[_nspec(H), _nspec(H)],
        out_shape=[jax.ShapeDtypeStruct((NPAD, H), jnp.float32),
                   jax.ShapeDtypeStruct((NPAD, H), jnp.int32)],
    )(aggr2, xh, w1na, w1nb, b1, w2, b2, g, beta, w1a, w1b)


def _node_step_last(aggr2, xh, w1na, w1nb, b1, w2, b2, g, beta,
                    wd1a, bd1a, wd2a, wd1b, bd1b, wd2b, bdec):
    def body(a_ref, xh_ref, w1na_ref, w1nb_ref, b1_ref, w2_ref, b2_ref,
             g_ref, be_ref, wd1a_ref, bd1a_ref, wd2a_ref, wd1b_ref, bd1b_ref,
             wd2b_ref, bdec_ref, out_ref):
        aggr = a_ref[0] + a_ref[1]
        xh_v = xh_ref[...]
        pre = _dot(aggr, w1na_ref[...]) + _dot(xh_v, w1nb_ref[...]) + b1_ref[...]
        h = jnp.maximum(pre, 0.0)
        y = _dot(h, w2_ref[...]) + b2_ref[...]
        xn = xh_v + _ln(y, g_ref[...], be_ref[...])
        h1 = jnp.maximum(_dot(xn, wd1a_ref[...]) + bd1a_ref[...], 0.0)
        h2 = jnp.maximum(_dot(xn, wd1b_ref[...]) + bd1b_ref[...], 0.0)
        out_ref[...] = _dot(h1, wd2a_ref[...]) + _dot(h2, wd2b_ref[...]) \
            + bdec_ref[...]

    aspec = pl.BlockSpec((2, NBLK, H), lambda i: (0, i, 0))
    return pl.pallas_call(
        body,
        grid=(NPAD // NBLK,),
        in_specs=[aspec, _nspec(H), _wspec(H, H), _wspec(H, H), _wspec(1, H),
                  _wspec(H, H), _wspec(1, H), _wspec(1, H), _wspec(1, H),
                  _wspec(H, H), _wspec(1, H), _wspec(H, H), _wspec(H, H),
                  _wspec(1, H), _wspec(H, H), _wspec(1, H)],
        out_specs=_nspec(H),
        out_shape=jax.ShapeDtypeStruct((NPAD, H), jnp.float32),
    )(aggr2, xh, w1na, w1nb, b1, w2, b2, g, beta,
      wd1a, bd1a, wd2a, wd1b, bd1b, wd2b, bdec)


def _row(v):
    return v.reshape(1, -1)


def _padn(a):
    return jnp.pad(a, ((0, NPAD - N), (0, 0)))


def _pack_jnp(cols):
    k = cols.shape[1] // 2
    u = lax.bitcast_convert_type(cols, jnp.int32)
    r = (u + 32767 + ((u >> 16) & 1)) >> 16
    return (r[:, :k] & 65535) | (r[:, k:] << 16)


def kernel(world_pos, mesh_pos, prev_world_pos, phi, prev_phi, swelling_phi,
           swelling_phi_rate, swelling_phi_rate_prev, node_type, mat_param,
           edge_index, params):
    f32 = jnp.float32
    src = edge_index[0].astype(jnp.int32).reshape(NW, NCHUNK, KC)
    dst = edge_index[1].astype(jnp.int32).reshape(NW, NCHUNK, KC)

    x = jnp.concatenate(
        [world_pos - prev_world_pos, phi - prev_phi, swelling_phi,
         swelling_phi_rate, swelling_phi_rate_prev, node_type], axis=-1)
    x = _padn(jnp.pad(x, ((0, 0), (0, 6)))).astype(f32)

    p32 = lax.bitcast_convert_type(_padn(jnp.pad(
        jnp.concatenate([mesh_pos, world_pos, phi], axis=-1),
        ((0, 0), (0, 11)))).astype(f32), jnp.int32)

    ne = params["node_encoder"]
    ee = params["edge_encoder"]
    proc = params["proc"]

    wn1 = jnp.pad(ne["W1"], ((0, 6), (0, 0)))
    we1 = jnp.pad(ee["W1"][jnp.array([0, 1, 3, 4, 6, 2, 5]), :],
                  ((0, 9), (0, 0)))

    ew = [p["edge_mlp"] for p in proc]
    nw_ = [p["node_mlp"] for p in proc]
    w1a = [w["W1"][:H] for w in ew]
    w1b = [w["W1"][H:2 * H] for w in ew]
    w1c = [w["W1"][2 * H:] for w in ew]

    xh, c = _encode_nodes(x, wn1, _row(ne["b1"]), ne["W2"], _row(ne["b2"]),
                          _row(ne["g"]), _row(ne["beta"]), w1a[0], w1b[0])

    zeros = jnp.zeros((NPAD, H), f32)
    wd = params["world_pos_decoder"]
    pdx = params["phi_decoder"]
    wd2a = jnp.pad(wd["W2"], ((0, 0), (0, H - 2)))
    wd2b = jnp.pad(pdx["W2"], ((0, 0), (2, H - 3)))
    bdec = _row(jnp.pad(jnp.concatenate([wd["b2"], pdx["b2"]]), (0, H - 3)))

    for i in range(3):
        e = ew[i]
        nm = nw_[i]
        if i == 0:
            t0 = jnp.concatenate(
                [c, p32, jnp.zeros((NPAD, 112), jnp.int32)], axis=1)
            g32 = _gather_combine(t0, dst, src, 144, True)
            msg, eh = _edge_step0(
                g32, we1, _row(ee["b1"]), ee["W2"], _row(ee["b2"]),
                _row(ee["g"]), _row(ee["beta"]), w1c[0], _row(e["b1"]),
                e["W2"], _row(e["b2"]), _row(e["g"]), _row(e["beta"]))
        else:
            g32 = _gather_combine(c, dst, src, H, False)
            msg, eh = _edge_step(eh, g32, w1c[i], _row(e["b1"]), e["W2"],
                                 _row(e["b2"]), _row(e["g"]), _row(e["beta"]))
        aggr2 = _scatter_add(msg, dst, zeros).reshape(2, NPAD, H)
        if i < 2:
            xh, c = _node_step(aggr2, xh, nm["W1"][:H], nm["W1"][H:],
                               _row(nm["b1"]), nm["W2"], _row(nm["b2"]),
                               _row(nm["g"]), _row(nm["beta"]),
                               w1a[i + 1], w1b[i + 1])
        else:
            out = _node_step_last(aggr2, xh, nm["W1"][:H], nm["W1"][H:],
                                  _row(nm["b1"]), nm["W2"], _row(nm["b2"]),
                                  _row(nm["g"]), _row(nm["beta"]),
                                  wd["W1"], _row(wd["b1"]), wd2a,
                                  pdx["W1"], _row(pdx["b1"]), wd2b, bdec)
    return out[:N, :3]
